# Initial kernel scaffold; baseline (speedup 1.0000x reference)
import jax
import jax.numpy as jnp
from jax.experimental import pallas as pl


def kernel(x1, edge_index1, x2, edge_index2, W1, b1, W2, b2):
    raise NotImplementedError("write your pallas kernel here")


# speedup vs baseline: 6.4115x; 6.4115x over previous
import functools

import jax
import jax.numpy as jnp
from jax import lax
from jax.experimental import pallas as pl
from jax.experimental.pallas import tpu as pltpu
from jax.experimental.pallas import tpu_sc as plsc

N = 10000
E = 160000
DIN = 256
HID = 512
FC = 128
NP = 10240
NS = 16
NC = 2
B = 128
NB = 79
EP = NS * NB * B
RPT = NP // NS
MT = 512

_MESH = plsc.VectorSubcoreMesh(
    core_axis_name="c", subcore_axis_name="s", num_cores=NC, num_subcores=NS
)


def _deg_body(dst1, dst2, dinv1, dinv2, hist_v, idx_v, red_v, out_v, shared):
    c = lax.axis_index("c")
    s = lax.axis_index("s")

    def run(dst_ref, out_ref):
        def zero(i, carry):
            hist_v[pl.ds(i * 16, 16)] = jnp.zeros((16,), jnp.float32)
            return carry

        lax.fori_loop(0, NP // 16, zero, 0)

        ones = jnp.ones((16,), jnp.float32)

        def batch(b, carry):
            base = (s * NB + b) * B
            pltpu.sync_copy(dst_ref.at[pl.ds(base, B)], idx_v)
            for j in range(B // 16):
                idx16 = idx_v[pl.ds(j * 16, 16)]
                plsc.addupdate_scatter(hist_v, [idx16], ones)
            return carry

        lax.fori_loop(0, NB, batch, 0)

        pltpu.sync_copy(hist_v, shared.at[s])
        plsc.subcore_barrier()
        pltpu.sync_copy(shared.at[:, pl.ds(s * RPT, RPT)], red_v)

        def col(j, carry):
            acc = red_v[0, pl.ds(j * 16, 16)]
            for r in range(1, NS):
                acc = acc + red_v[r, pl.ds(j * 16, 16)]
            d = acc + 1.0
            yi = jnp.full((16,), 0x5F3759DF, jnp.int32) - lax.shift_right_logical(
                plsc.bitcast(d, jnp.int32), 1
            )
            y = plsc.bitcast(yi, jnp.float32)
            for _ in range(3):
                y = y * (1.5 - 0.5 * d * y * y)
            out_v[pl.ds(j * 16, 16)] = y
            return carry

        lax.fori_loop(0, RPT // 16, col, 0)
        pltpu.sync_copy(out_v, out_ref.at[pl.ds(s * RPT, RPT)])

    @pl.when(c == 0)
    def _():
        run(dst1, dinv1)

    @pl.when(c == 1)
    def _():
        run(dst2, dinv2)


_deg_kernel = pl.kernel(
    _deg_body,
    out_type=[jax.ShapeDtypeStruct((NP,), jnp.float32)] * 2,
    mesh=_MESH,
    compiler_params=pltpu.CompilerParams(needs_layout_passes=False),
    scratch_types=[
        pltpu.VMEM((NP,), jnp.float32),
        pltpu.VMEM((B,), jnp.int32),
        pltpu.VMEM((NS, RPT), jnp.float32),
        pltpu.VMEM((RPT,), jnp.float32),
        pltpu.VMEM_SHARED((NS, NP), jnp.float32),
    ],
)


def _make_agg_body(npc):
    nch = npc * NC

    def body(*refs):
        xs = refs[0:nch]
        srcp = refs[nch]
        dstp = refs[nch + 1]
        outs = refs[nch + 2 : 2 * nch + 2]
        acc, idx_s, idx_d, rows, sem = refs[2 * nch + 2 :]
        c = lax.axis_index("c")
        s = lax.axis_index("s")

        def proc(xs_ref, out_ref):
            pltpu.sync_copy(
                xs_ref.at[pl.ds(s * RPT, RPT)], acc.at[pl.ds(s * RPT, RPT)]
            )
            plsc.subcore_barrier()

            def batch(b, carry):
                base = (s * NB + b) * B
                pltpu.sync_copy(srcp.at[pl.ds(base, B)], idx_s)
                pltpu.sync_copy(dstp.at[pl.ds(base, B)], idx_d)
                pltpu.async_copy(xs_ref.at[idx_s], rows, sem).wait()
                pltpu.sync_copy(rows, acc.at[idx_d], add=True)
                return carry

            lax.fori_loop(0, NB, batch, 0)
            plsc.subcore_barrier()
            pltpu.sync_copy(
                acc.at[pl.ds(s * RPT, RPT)], out_ref.at[pl.ds(s * RPT, RPT)]
            )
            plsc.subcore_barrier()

        for k in range(npc):

            @pl.when(c == 0)
            def _():
                proc(xs[k], outs[k])

            @pl.when(c == 1)
            def _():
                proc(xs[npc + k], outs[npc + k])

    return body


def _make_agg_kernel(npc):
    nch = npc * NC
    return pl.kernel(
        _make_agg_body(npc),
        out_type=[jax.ShapeDtypeStruct((NP, FC), jnp.float32)] * nch,
        mesh=_MESH,
        scratch_types=[
            pltpu.VMEM_SHARED((NP, FC), jnp.float32),
            pltpu.VMEM((B,), jnp.int32),
            pltpu.VMEM((B,), jnp.int32),
            pltpu.VMEM((B, FC), jnp.float32),
            pltpu.SemaphoreType.DMA,
        ],
    )


_agg1 = _make_agg_kernel(1)
_agg2 = _make_agg_kernel(2)


def _scale_kernel_body(x_ref, dv_ref, o0_ref, o1_ref):
    xs = x_ref[...] * dv_ref[...]
    o0_ref[...] = xs[:, :FC]
    o1_ref[...] = xs[:, FC:]


@functools.partial(jax.jit)
def _scale(x_pad, dv):
    return pl.pallas_call(
        _scale_kernel_body,
        grid=(NP // MT,),
        in_specs=[
            pl.BlockSpec((MT, DIN), lambda i: (i, 0)),
            pl.BlockSpec((MT, 1), lambda i: (i, 0)),
        ],
        out_specs=[
            pl.BlockSpec((MT, FC), lambda i: (i, 0)),
            pl.BlockSpec((MT, FC), lambda i: (i, 0)),
        ],
        out_shape=[jax.ShapeDtypeStruct((NP, FC), jnp.float32)] * 2,
    )(x_pad, dv)


def _mm1_body(s0_ref, s1_ref, w_ref, b_ref, dv_ref, o0, o1, o2, o3):
    acc = jnp.dot(s0_ref[...], w_ref[:FC, :], preferred_element_type=jnp.float32)
    acc = acc + jnp.dot(
        s1_ref[...], w_ref[FC:, :], preferred_element_type=jnp.float32
    )
    dv = dv_ref[...]
    h = jnp.maximum(acc * dv + b_ref[...], 0.0) * dv
    o0[...] = h[:, 0:FC]
    o1[...] = h[:, FC : 2 * FC]
    o2[...] = h[:, 2 * FC : 3 * FC]
    o3[...] = h[:, 3 * FC : 4 * FC]


@functools.partial(jax.jit)
def _mm1(s0, s1, w1, b1, dv):
    return pl.pallas_call(
        _mm1_body,
        grid=(NP // MT,),
        in_specs=[
            pl.BlockSpec((MT, FC), lambda i: (i, 0)),
            pl.BlockSpec((MT, FC), lambda i: (i, 0)),
            pl.BlockSpec((DIN, HID), lambda i: (0, 0)),
            pl.BlockSpec((1, HID), lambda i: (0, 0)),
            pl.BlockSpec((MT, 1), lambda i: (i, 0)),
        ],
        out_specs=[pl.BlockSpec((MT, FC), lambda i: (i, 0))] * 4,
        out_shape=[jax.ShapeDtypeStruct((NP, FC), jnp.float32)] * 4,
    )(s0, s1, w1, b1, dv)


def _mm2_body(s0_ref, s1_ref, s2_ref, s3_ref, w_ref, b_ref, dv_ref, o_ref):
    acc = jnp.dot(s0_ref[...], w_ref[:FC, :], preferred_element_type=jnp.float32)
    acc = acc + jnp.dot(
        s1_ref[...], w_ref[FC : 2 * FC, :], preferred_element_type=jnp.float32
    )
    acc = acc + jnp.dot(
        s2_ref[...], w_ref[2 * FC : 3 * FC, :], preferred_element_type=jnp.float32
    )
    acc = acc + jnp.dot(
        s3_ref[...], w_ref[3 * FC :, :], preferred_element_type=jnp.float32
    )
    o_ref[...] = acc * dv_ref[...] + b_ref[...]


@functools.partial(jax.jit)
def _mm2(s0, s1, s2, s3, w2, b2, dv):
    return pl.pallas_call(
        _mm2_body,
        grid=(NP // MT,),
        in_specs=[
            pl.BlockSpec((MT, FC), lambda i: (i, 0)),
            pl.BlockSpec((MT, FC), lambda i: (i, 0)),
            pl.BlockSpec((MT, FC), lambda i: (i, 0)),
            pl.BlockSpec((MT, FC), lambda i: (i, 0)),
            pl.BlockSpec((HID, HID), lambda i: (0, 0)),
            pl.BlockSpec((1, HID), lambda i: (0, 0)),
            pl.BlockSpec((MT, 1), lambda i: (i, 0)),
        ],
        out_specs=pl.BlockSpec((MT, HID), lambda i: (i, 0)),
        out_shape=jax.ShapeDtypeStruct((NP, HID), jnp.float32),
    )(s0, s1, s2, s3, w2, b2, dv)


def _view(x, srcp, dstp, dv, W1, b1r, W2, b2r):
    x_pad = jnp.pad(x, ((0, NP - N), (0, 0)))
    xs0, xs1 = _scale(x_pad, dv)
    s0, s1 = _agg1(xs0, xs1, srcp, dstp)
    h0, h1, h2, h3 = _mm1(s0, s1, W1, b1r, dv)
    t0, t1, t2, t3 = _agg2(h0, h1, h2, h3, srcp, dstp)
    z = _mm2(t0, t1, t2, t3, W2, b2r, dv)
    return z[:N]


def kernel(x1, edge_index1, x2, edge_index2, W1, b1, W2, b2):
    pad_src = jnp.zeros((EP - E,), jnp.int32)
    pad_dst = jnp.full((EP - E,), N, jnp.int32)
    src1 = jnp.concatenate([edge_index1[0], pad_src])
    dst1 = jnp.concatenate([edge_index1[1], pad_dst])
    src2 = jnp.concatenate([edge_index2[0], pad_src])
    dst2 = jnp.concatenate([edge_index2[1], pad_dst])

    dinv1, dinv2 = _deg_kernel(dst1, dst2)
    dv1 = dinv1.reshape(NP, 1)
    dv2 = dinv2.reshape(NP, 1)
    b1r = b1.reshape(1, HID)
    b2r = b2.reshape(1, HID)

    z1 = _view(x1, src1, dst1, dv1, W1, b1r, W2, b2r)
    z2 = _view(x2, src2, dst2, dv2, W1, b1r, W2, b2r)
    return (z1, z2)
